# baseline (speedup 1.0000x reference)
import functools

import jax
import jax.numpy as jnp
from jax import lax
from jax.experimental import pallas as pl
from jax.experimental.pallas import tpu as pltpu
from jax.experimental.pallas import tpu_sc as plsc

NUM_ACTIONS = 7
EMBED_DIM = 64
QUAD = 4
QROW = QUAD * EMBED_DIM
NQT = NUM_ACTIONS ** QUAD

NC = 2
NS = 16
NW = NC * NS
L = 16

CH = 128
CQ = CH // QUAD
NBUF = 8
NBUF_T = 9


@functools.partial(jax.jit, static_argnums=(3,))
def _lookup(table, qtable, idx, B):
    b_per_w = B // NW
    q_per_w = b_per_w // QUAD
    nchunk = b_per_w // CH
    ngroups = nchunk // NBUF
    mesh = plsc.VectorSubcoreMesh(core_axis_name="c", subcore_axis_name="s")

    @functools.partial(
        pl.kernel,
        out_type=jax.ShapeDtypeStruct((B // QUAD, QROW), jnp.float32),
        mesh=mesh,
        compiler_params=pltpu.CompilerParams(
            use_tc_tiling_on_sc=False, needs_layout_passes=False),
        scratch_types=[
            pltpu.VMEM((NUM_ACTIONS * EMBED_DIM,), jnp.float32),
            pltpu.VMEM((b_per_w,), jnp.int32),
            pltpu.VMEM((q_per_w,), jnp.int32),
            pltpu.VMEM((NBUF_T, CQ, QROW), jnp.float32),
            [pltpu.SemaphoreType.DMA] * 2,
            [pltpu.SemaphoreType.DMA] * NBUF_T,
        ],
    )
    def lookup(table_hbm, qtable_hbm, idx_hbm, out_hbm, table_v, idx_v,
               qidx_v, bufs, gsems, ssems):
        wid = lax.axis_index("s") * NC + lax.axis_index("c")
        base = wid * b_per_w
        qbase = wid * q_per_w
        pltpu.sync_copy(table_hbm, table_v)
        pltpu.sync_copy(idx_hbm.at[pl.ds(base, b_per_w)], idx_v)

        iota = lax.iota(jnp.int32, L)

        def pack_quads(c0):
            for k in range(CQ // L):
                p = c0 * CQ // L + k
                posv = (p * L + iota) * QUAD
                q = plsc.load_gather(idx_v, [posv])
                for j in range(1, QUAD):
                    q = q * NUM_ACTIONS + plsc.load_gather(idx_v, [posv + j])
                qidx_v[pl.ds(p * L, L)] = q

        def build(c, b):
            def grp(g, carry):
                a_vec = idx_v[pl.ds(c * CH + g * L, L)]
                src = a_vec * EMBED_DIM
                rows = g * L + iota
                qrow = lax.shift_right_logical(rows, 2)
                dst0 = (rows & (QUAD - 1)) * EMBED_DIM
                @plsc.parallel_loop(0, EMBED_DIM, unroll=16)
                def col(cc):
                    colv = (iota + cc) & (EMBED_DIM - 1)
                    v = plsc.load_gather(table_v, [src + colv])
                    plsc.store_scatter(bufs.at[b], [qrow, dst0 + colv], v)
                return carry
            lax.fori_loop(0, CH // L, grp, 0)

        def gather(c, d):
            return pltpu.make_async_copy(
                qtable_hbm.at[qidx_v.at[pl.ds(c * CQ, CQ)]],
                bufs.at[8 * d if d else 0], gsems[d])

        def scatter(c, b):
            return pltpu.make_async_copy(
                bufs.at[b], out_hbm.at[pl.ds(qbase + c * CQ, CQ)], ssems[b])

        def group_block(g, d, prologue=False, epilogue=False):
            c0 = g * NBUF
            db = 8 * d if d else 0
            gather(c0, d).wait()
            scatter(c0, db).start()
            for b in range(1, NBUF):
                c = c0 + b
                if not prologue:
                    scatter(c - NBUF, b).wait()
                build(c, b)
                scatter(c, b).start()
            scatter(c0, db).wait()
            if not epilogue:
                pack_quads(c0 + 2 * NBUF)
                gather(c0 + 2 * NBUF, d).start()

        pack_quads(0)
        pack_quads(NBUF)
        gather(0, 0).start()
        gather(NBUF, 1).start()
        group_block(0, 0, prologue=True)
        group_block(1, 1)

        def pair(gp, carry):
            group_block(2 * gp + 2, 0)
            group_block(2 * gp + 3, 1)
            return carry

        lax.fori_loop(0, (ngroups - 5) // 2, pair, 0)
        group_block(ngroups - 3, 0)
        group_block(ngroups - 2, 1, epilogue=True)
        group_block(ngroups - 1, 0, epilogue=True)
        for b in range(1, NBUF):
            scatter((ngroups - 1) * NBUF + b, b).wait()

    return lookup(table, qtable, idx)


def _quad_table(rolled):
    n, d = rolled.shape
    parts = []
    for k in range(QUAD):
        shape = [1] * QUAD + [d]
        shape[k] = n
        parts.append(jnp.broadcast_to(
            rolled.reshape(shape), (n,) * QUAD + (d,)))
    return jnp.concatenate(parts, axis=-1).reshape(n ** QUAD, QUAD * d)


def kernel(action, action_embeddings):
    BATCH, HIST = action.shape
    B = BATCH * HIST
    rolled = jnp.roll(action_embeddings, 1, axis=0)
    out = _lookup(rolled.reshape(-1), _quad_table(rolled),
                  action.reshape(B), B)
    return out.reshape(BATCH, HIST, EMBED_DIM)


# speedup vs baseline: 1.0031x; 1.0031x over previous
import functools

import jax
import jax.numpy as jnp
from jax import lax
from jax.experimental import pallas as pl
from jax.experimental.pallas import tpu as pltpu
from jax.experimental.pallas import tpu_sc as plsc

NUM_ACTIONS = 7
EMBED_DIM = 64
QUAD = 4
QROW = QUAD * EMBED_DIM
NQT = NUM_ACTIONS ** QUAD

NC = 2
NS = 16
NW = NC * NS
L = 16

CH = 128
CQ = CH // QUAD
NBUF = 8
NBUF_T = 9


@functools.partial(jax.jit, static_argnums=(3,))
def _lookup(table, qtable, idx, B):
    b_per_w = B // NW
    q_per_w = b_per_w // QUAD
    nchunk = b_per_w // CH
    ngroups = nchunk // NBUF
    mesh = plsc.VectorSubcoreMesh(core_axis_name="c", subcore_axis_name="s")

    @functools.partial(
        pl.kernel,
        out_type=jax.ShapeDtypeStruct((B // QUAD, QROW), jnp.float32),
        mesh=mesh,
        compiler_params=pltpu.CompilerParams(
            use_tc_tiling_on_sc=False, needs_layout_passes=False),
        scratch_types=[
            pltpu.VMEM((NUM_ACTIONS * EMBED_DIM,), jnp.float32),
            pltpu.VMEM((b_per_w,), jnp.int32),
            pltpu.VMEM((q_per_w,), jnp.int32),
            pltpu.VMEM((NBUF_T, CQ, QROW), jnp.float32),
            [pltpu.SemaphoreType.DMA] * 2,
            [pltpu.SemaphoreType.DMA] * NBUF_T,
        ],
    )
    def lookup(table_hbm, qtable_hbm, idx_hbm, out_hbm, table_v, idx_v,
               qidx_v, bufs, gsems, ssems):
        wid = lax.axis_index("s") * NC + lax.axis_index("c")
        base = wid * b_per_w
        qbase = wid * q_per_w
        pltpu.sync_copy(table_hbm, table_v)
        pltpu.sync_copy(idx_hbm.at[pl.ds(base, b_per_w)], idx_v)

        iota = lax.iota(jnp.int32, L)

        def pack_quads(c0):
            for k in range(CQ // L):
                p = c0 * CQ // L + k
                posv = (p * L + iota) * QUAD
                q = plsc.load_gather(idx_v, [posv])
                for j in range(1, QUAD):
                    q = q * NUM_ACTIONS + plsc.load_gather(idx_v, [posv + j])
                qidx_v[pl.ds(p * L, L)] = q

        qcols = [iota + 16 * j for j in range(EMBED_DIM // L)]

        def build(c, b):
            @plsc.parallel_loop(0, CH, unroll=8)
            def row(r):
                pos = jnp.broadcast_to(c * CH + r, (L,)).astype(jnp.int32)
                srcb = plsc.load_gather(idx_v, [pos]) * EMBED_DIM
                qrow = lax.shift_right_logical(r, 2)
                dst0 = (r & (QUAD - 1)) * EMBED_DIM
                for j in range(EMBED_DIM // L):
                    v = plsc.load_gather(table_v, [srcb + qcols[j]])
                    bufs[b, qrow, pl.ds(dst0 + 16 * j, L)] = v

        def gather(c, d):
            return pltpu.make_async_copy(
                qtable_hbm.at[qidx_v.at[pl.ds(c * CQ, CQ)]],
                bufs.at[8 * d if d else 0], gsems[d])

        def scatter(c, b):
            return pltpu.make_async_copy(
                bufs.at[b], out_hbm.at[pl.ds(qbase + c * CQ, CQ)], ssems[b])

        def group_block(g, d, prologue=False, epilogue=False):
            c0 = g * NBUF
            db = 8 * d if d else 0
            gather(c0, d).wait()
            scatter(c0, db).start()
            for b in range(1, NBUF):
                c = c0 + b
                if not prologue:
                    scatter(c - NBUF, b).wait()
                build(c, b)
                scatter(c, b).start()
            scatter(c0, db).wait()
            if not epilogue:
                pack_quads(c0 + 2 * NBUF)
                gather(c0 + 2 * NBUF, d).start()

        pack_quads(0)
        pack_quads(NBUF)
        gather(0, 0).start()
        gather(NBUF, 1).start()
        group_block(0, 0, prologue=True)
        group_block(1, 1)

        def pair(gp, carry):
            group_block(2 * gp + 2, 0)
            group_block(2 * gp + 3, 1)
            return carry

        lax.fori_loop(0, (ngroups - 5) // 2, pair, 0)
        group_block(ngroups - 3, 0)
        group_block(ngroups - 2, 1, epilogue=True)
        group_block(ngroups - 1, 0, epilogue=True)
        for b in range(1, NBUF):
            scatter((ngroups - 1) * NBUF + b, b).wait()

    return lookup(table, qtable, idx)


def _quad_table(rolled):
    n, d = rolled.shape
    parts = []
    for k in range(QUAD):
        shape = [1] * QUAD + [d]
        shape[k] = n
        parts.append(jnp.broadcast_to(
            rolled.reshape(shape), (n,) * QUAD + (d,)))
    return jnp.concatenate(parts, axis=-1).reshape(n ** QUAD, QUAD * d)


def kernel(action, action_embeddings):
    BATCH, HIST = action.shape
    B = BATCH * HIST
    rolled = jnp.roll(action_embeddings, 1, axis=0)
    out = _lookup(rolled.reshape(-1), _quad_table(rolled),
                  action.reshape(B), B)
    return out.reshape(BATCH, HIST, EMBED_DIM)
